# baseline (speedup 1.0000x reference)
import functools

import jax
import jax.numpy as jnp
from jax import lax
from jax.experimental import pallas as pl
from jax.experimental.pallas import tpu as pltpu
from jax.experimental.pallas import tpu_sc as plsc

_NC = 2
_NS = 16
_L = 16
_CR = 8
_CC = 4096
_NB = 3


@functools.lru_cache(maxsize=None)
def _make_sc_diag_scatter(n, diag_len):
    off = n - diag_len
    nw = _NC * _NS
    rw = n // nw
    ncc = n // _CC
    nchunks = (rw // _CR) * ncc

    mesh = plsc.VectorSubcoreMesh(
        core_axis_name="c", subcore_axis_name="s",
        num_cores=_NC, num_subcores=_NS,
    )

    @functools.partial(
        pl.kernel,
        out_type=jax.ShapeDtypeStruct((n, n), jnp.float32),
        mesh=mesh,
        scratch_types=[
            pltpu.VMEM((_NB, _CR, _CC), jnp.float32),
            pltpu.VMEM((rw + 2 * _L,), jnp.float32),
            pltpu.SemaphoreType.DMA((_NB,)),
            pltpu.SemaphoreType.DMA((_NB,)),
        ],
    )
    def sc_kernel(x_hbm, src_hbm, out_hbm, buf, s_v, in_sem, out_sem):
        wid = lax.axis_index("s") * _NC + lax.axis_index("c")
        base = wid * rw
        pltpu.sync_copy(src_hbm.at[pl.ds(base, rw)], s_v.at[pl.ds(_L, rw)])

        def slab(c):
            k, cc = divmod(c, ncc)
            return (pl.ds(base + k * _CR, _CR), pl.ds(cc * _CC, _CC))

        def chunk_in(c, b, do_wait):
            cp = pltpu.make_async_copy(x_hbm.at[slab(c)], buf.at[b], in_sem.at[b])
            cp.wait() if do_wait else cp.start()

        def chunk_out(c, b, do_wait):
            cp = pltpu.make_async_copy(buf.at[b], out_hbm.at[slab(c)], out_sem.at[b])
            cp.wait() if do_wait else cp.start()

        def patch(c, b):
            k, cc = divmod(c, ncc)
            for i in range(_CR):
                r = k * _CR + i
                g = base + r
                l = (r + off) % _L
                gc = g + off
                a = gc - l - cc * _CC
                cond = (a >= 0) & (a < _CC) & (g < diag_len)
                a_s = pl.multiple_of(jnp.clip(a, 0, _CC - _L), _L)
                val = s_v[pl.ds(_L + r - l, _L)]
                cf = cond.astype(jnp.float32)
                mask = lax.iota(jnp.int32, _L) == l
                cur = buf[b, i, pl.ds(a_s, _L)]
                buf[b, i, pl.ds(a_s, _L)] = jnp.where(
                    mask, cur + cf * (val - cur), cur)

        for pr in range(_NB - 1):
            chunk_in(pr, pr, False)
        for c in range(nchunks):
            b = c % _NB
            p = c + _NB - 1
            if p < nchunks:
                if c >= 1:
                    chunk_out(c - 1, (c - 1) % _NB, True)
                chunk_in(p, p % _NB, False)
            chunk_in(c, b, True)
            patch(c, b)
            chunk_out(c, b, False)
        for t in range(max(0, nchunks - _NB), nchunks):
            chunk_out(t, t % _NB, True)

    return sc_kernel


def kernel(x, src, offset, dim1, dim2):
    n = x.shape[0]
    diag_len = src.shape[0]
    src_pad = jnp.pad(src, (0, n - diag_len))
    return _make_sc_diag_scatter(n, diag_len)(x, src_pad)


# speedup vs baseline: 1.0067x; 1.0067x over previous
import functools

import jax
import jax.numpy as jnp
from jax import lax
from jax.experimental import pallas as pl
from jax.experimental.pallas import tpu as pltpu
from jax.experimental.pallas import tpu_sc as plsc

_NC = 2
_NS = 16
_L = 16
_CR = 8
_CC = 4096
_NB = 3


@functools.lru_cache(maxsize=None)
def _make_sc_diag_scatter(n, diag_len):
    off = n - diag_len
    nw = _NC * _NS
    rw = n // nw
    ncc = n // _CC
    nchunks = (rw // _CR) * ncc

    mesh = plsc.VectorSubcoreMesh(
        core_axis_name="c", subcore_axis_name="s",
        num_cores=_NC, num_subcores=_NS,
    )

    @functools.partial(
        pl.kernel,
        out_type=jax.ShapeDtypeStruct((n, n), jnp.float32),
        mesh=mesh,
        scratch_types=[
            pltpu.VMEM((_NB, _CR, _CC), jnp.float32),
            pltpu.VMEM((rw + 2 * _L,), jnp.float32),
            pltpu.SemaphoreType.DMA((_NB,)),
            pltpu.SemaphoreType.DMA((_NB,)),
        ],
    )
    def sc_kernel(x_hbm, src_hbm, out_hbm, buf, s_v, in_sem, out_sem):
        wid = lax.axis_index("s") * _NC + lax.axis_index("c")
        base = wid * rw

        def slab(c):
            k, cc = divmod(c, ncc)
            return (pl.ds(base + k * _CR, _CR), pl.ds(cc * _CC, _CC))

        def chunk_in(c, b, do_wait):
            cp = pltpu.make_async_copy(x_hbm.at[slab(c)], buf.at[b], in_sem.at[b])
            cp.wait() if do_wait else cp.start()

        def chunk_out(c, b, do_wait):
            cp = pltpu.make_async_copy(buf.at[b], out_hbm.at[slab(c)], out_sem.at[b])
            cp.wait() if do_wait else cp.start()

        def patch_rows(c, b):
            k, cc = divmod(c, ncc)
            for i in range(_CR):
                r = k * _CR + i
                g = base + r
                l = (r + off) % _L
                gc = g + off
                a = gc - l - cc * _CC
                cond = (a >= 0) & (a < _CC) & (g < diag_len)
                a_s = pl.multiple_of(jnp.clip(a, 0, _CC - _L), _L)
                val = s_v[pl.ds(_L + r - l, _L)]
                cf = cond.astype(jnp.float32)
                mask = lax.iota(jnp.int32, _L) == l
                cur = buf[b, i, pl.ds(a_s, _L)]
                buf[b, i, pl.ds(a_s, _L)] = jnp.where(
                    mask, cur + cf * (val - cur), cur)

        def patch(c, b):
            k, cc = divmod(c, ncc)
            glo = base + k * _CR + off
            ghi = glo + _CR
            hit = (ghi > cc * _CC) & (glo < (cc + 1) * _CC)

            @pl.when(hit)
            def _():
                patch_rows(c, b)

        for pr in range(_NB - 1):
            chunk_in(pr, pr, False)
        pltpu.sync_copy(src_hbm.at[pl.ds(base, rw)], s_v.at[pl.ds(_L, rw)])
        for c in range(nchunks):
            b = c % _NB
            p = c + _NB - 1
            if p < nchunks:
                if c >= 1:
                    chunk_out(c - 1, (c - 1) % _NB, True)
                chunk_in(p, p % _NB, False)
            chunk_in(c, b, True)
            patch(c, b)
            chunk_out(c, b, False)
        for t in range(max(0, nchunks - _NB), nchunks):
            chunk_out(t, t % _NB, True)

    return sc_kernel


def kernel(x, src, offset, dim1, dim2):
    n = x.shape[0]
    diag_len = src.shape[0]
    src_pad = jnp.pad(src, (0, n - diag_len))
    return _make_sc_diag_scatter(n, diag_len)(x, src_pad)


# speedup vs baseline: 1.0160x; 1.0092x over previous
import functools

import jax
import jax.numpy as jnp
from jax import lax
from jax.experimental import pallas as pl
from jax.experimental.pallas import tpu as pltpu
from jax.experimental.pallas import tpu_sc as plsc

_NC = 2
_NS = 16
_L = 16
_CR = 8
_CC = 4096
_NB = 3


@functools.lru_cache(maxsize=None)
def _make_sc_diag_scatter(n, diag_len):
    off = n - diag_len
    nw = _NC * _NS
    rw = n // nw
    ncc = n // _CC
    nchunks = (rw // _CR) * ncc

    mesh = plsc.VectorSubcoreMesh(
        core_axis_name="c", subcore_axis_name="s",
        num_cores=_NC, num_subcores=_NS,
    )

    @functools.partial(
        pl.kernel,
        out_type=jax.ShapeDtypeStruct((n, n), jnp.float32),
        mesh=mesh,
        scratch_types=[
            pltpu.VMEM((_NB, _CR, _CC), jnp.float32),
            pltpu.VMEM((rw + 2 * _L,), jnp.float32),
            pltpu.SemaphoreType.DMA((_NB,)),
            pltpu.SemaphoreType.DMA((_NB,)),
        ],
    )
    def sc_kernel(x_hbm, src_hbm, out_hbm, buf, s_v, in_sem, out_sem):
        wid = lax.axis_index("s") * _NC + lax.axis_index("c")
        base = wid * rw

        def slab(c):
            k, cc = divmod(c, ncc)
            return (pl.ds(base + k * _CR, _CR), pl.ds(cc * _CC, _CC))

        def chunk_in(c, b, do_wait):
            cp = pltpu.make_async_copy(x_hbm.at[slab(c)], buf.at[b], in_sem.at[b])
            cp.wait() if do_wait else cp.start()

        def chunk_out(c, b, do_wait):
            cp = pltpu.make_async_copy(buf.at[b], out_hbm.at[slab(c)], out_sem.at[b])
            cp.wait() if do_wait else cp.start()

        def patch_rows(c, b):
            k, cc = divmod(c, ncc)
            for i in range(_CR):
                r = k * _CR + i
                g = base + r
                l = (r + off) % _L
                gc = g + off
                a = gc - l - cc * _CC
                cond = (a >= 0) & (a < _CC) & (g < diag_len)
                a_s = pl.multiple_of(jnp.clip(a, 0, _CC - _L), _L)
                val = s_v[pl.ds(_L + r - l, _L)]
                mask = lax.iota(jnp.int32, _L) == l
                cur = buf[b, i, pl.ds(a_s, _L)]
                m = jnp.where(
                    mask,
                    jnp.full((_L,), -1, jnp.int32),
                    jnp.full((_L,), 0, jnp.int32),
                ) * cond.astype(jnp.int32)
                vb = lax.bitcast_convert_type(val, jnp.int32)
                cb = lax.bitcast_convert_type(cur, jnp.int32)
                nb = cb ^ ((cb ^ vb) & m)
                buf[b, i, pl.ds(a_s, _L)] = lax.bitcast_convert_type(
                    nb, jnp.float32)

        def patch(c, b):
            k, cc = divmod(c, ncc)
            glo = base + k * _CR + off
            ghi = glo + _CR
            hit = (ghi > cc * _CC) & (glo < (cc + 1) * _CC)

            @pl.when(hit)
            def _():
                patch_rows(c, b)

        for pr in range(_NB - 1):
            chunk_in(pr, pr, False)
        pltpu.sync_copy(src_hbm.at[pl.ds(base, rw)], s_v.at[pl.ds(_L, rw)])
        for c in range(nchunks):
            b = c % _NB
            p = c + _NB - 1
            if p < nchunks:
                if c >= 1:
                    chunk_out(c - 1, (c - 1) % _NB, True)
                chunk_in(p, p % _NB, False)
            chunk_in(c, b, True)
            patch(c, b)
            chunk_out(c, b, False)
        for t in range(max(0, nchunks - _NB), nchunks):
            chunk_out(t, t % _NB, True)

    return sc_kernel


def kernel(x, src, offset, dim1, dim2):
    n = x.shape[0]
    diag_len = src.shape[0]
    src_pad = jnp.pad(src, (0, n - diag_len))
    return _make_sc_diag_scatter(n, diag_len)(x, src_pad)


# speedup vs baseline: 1.0345x; 1.0182x over previous
import functools

import jax
import jax.numpy as jnp
from jax import lax
from jax.experimental import pallas as pl
from jax.experimental.pallas import tpu as pltpu
from jax.experimental.pallas import tpu_sc as plsc

_NC = 2
_NS = 16
_L = 16
_BK = 16
_CR = 8
_CC = 4096
_NB = 3


@functools.lru_cache(maxsize=None)
def _make_sc_diag_scatter(n, diag_len):
    off = n - diag_len
    nw = _NC * _NS
    rw = n // nw
    ncc = n // _CC
    nj = rw // _BK
    nsub = (_BK // _CR) * ncc
    nchunks = nj * nsub

    mesh = plsc.VectorSubcoreMesh(
        core_axis_name="c", subcore_axis_name="s",
        num_cores=_NC, num_subcores=_NS,
    )

    @functools.partial(
        pl.kernel,
        out_type=jax.ShapeDtypeStruct((n, n), jnp.float32),
        mesh=mesh,
        scratch_types=[
            pltpu.VMEM((_NB, _CR, _CC), jnp.float32),
            pltpu.VMEM((rw + 2 * _L,), jnp.float32),
            pltpu.SemaphoreType.DMA((_NB,)),
            pltpu.SemaphoreType.DMA((_NB,)),
        ],
    )
    def sc_kernel(x_hbm, src_hbm, out_hbm, buf, s_v, in_sem, out_sem):
        wid = lax.axis_index("s") * _NC + lax.axis_index("c")
        base = wid * rw

        def parts(c):
            j, sub = divmod(c, nsub)
            h, cc = divmod(sub, ncc)
            row0 = (wid + j * nw) * _BK + h * _CR
            return j, h, cc, row0

        def slab(c):
            _, _, cc, row0 = parts(c)
            return (pl.ds(pl.multiple_of(row0, _CR), _CR),
                    pl.ds(cc * _CC, _CC))

        def chunk_in(c, b, do_wait):
            cp = pltpu.make_async_copy(x_hbm.at[slab(c)], buf.at[b], in_sem.at[b])
            cp.wait() if do_wait else cp.start()

        def chunk_out(c, b, do_wait):
            cp = pltpu.make_async_copy(buf.at[b], out_hbm.at[slab(c)], out_sem.at[b])
            cp.wait() if do_wait else cp.start()

        def patch_rows(c, b):
            j, h, cc, row0 = parts(c)
            for i in range(_CR):
                r = j * _BK + h * _CR + i
                g = row0 + i
                l = (h * _CR + i + off) % _L
                gc = g + off
                a = gc - l - cc * _CC
                cond = (a >= 0) & (a < _CC) & (g < diag_len)
                a_s = pl.multiple_of(jnp.clip(a, 0, _CC - _L), _L)
                val = s_v[pl.ds(_L + r - l, _L)]
                mask = lax.iota(jnp.int32, _L) == l
                cur = buf[b, i, pl.ds(a_s, _L)]
                m = jnp.where(
                    mask,
                    jnp.full((_L,), -1, jnp.int32),
                    jnp.full((_L,), 0, jnp.int32),
                ) * cond.astype(jnp.int32)
                vb = lax.bitcast_convert_type(val, jnp.int32)
                cb = lax.bitcast_convert_type(cur, jnp.int32)
                nb = cb ^ ((cb ^ vb) & m)
                buf[b, i, pl.ds(a_s, _L)] = lax.bitcast_convert_type(
                    nb, jnp.float32)

        def patch(c, b):
            _, _, cc, row0 = parts(c)
            glo = row0 + off
            hit = (glo + _CR > cc * _CC) & (glo < (cc + 1) * _CC)

            @pl.when(hit)
            def _():
                patch_rows(c, b)

        for pr in range(_NB - 1):
            chunk_in(pr, pr, False)
        pltpu.sync_copy(src_hbm.at[pl.ds(base, rw)], s_v.at[pl.ds(_L, rw)])
        for c in range(nchunks):
            b = c % _NB
            p = c + _NB - 1
            if p < nchunks:
                if c >= 1:
                    chunk_out(c - 1, (c - 1) % _NB, True)
                chunk_in(p, p % _NB, False)
            chunk_in(c, b, True)
            patch(c, b)
            chunk_out(c, b, False)
        for t in range(max(0, nchunks - _NB), nchunks):
            chunk_out(t, t % _NB, True)

    return sc_kernel


def kernel(x, src, offset, dim1, dim2):
    n = x.shape[0]
    diag_len = src.shape[0]
    nw = _NC * _NS
    nj = n // (nw * _BK)
    src_pad = jnp.pad(src, (0, n - diag_len))
    src_arr = src_pad.reshape(nj, nw, _BK).transpose(1, 0, 2).reshape(-1)
    return _make_sc_diag_scatter(n, diag_len)(x, src_arr)
